# baseline (speedup 1.0000x reference)
import jax
import jax.numpy as jnp
from jax import lax
from jax.experimental import pallas as pl
from jax.experimental.pallas import tpu as pltpu
from jax.experimental.pallas import tpu_sc as plsc

G = 128
D = 32
L = 16
MASK = (1 << 21) - 1
_P1 = 2654435761
_P2 = 805459861

_OFFS = [(0, 0, 0), (1, 0, 0), (0, 1, 0), (1, 1, 0),
         (0, 0, 1), (1, 0, 1), (0, 1, 1), (1, 1, 1)]


def _as_i32(v):
    v &= 0xFFFFFFFF
    return v - (1 << 32) if v >= (1 << 31) else v


_P1_I32 = _as_i32(_P1)
_P2_I32 = _as_i32(_P2)
_CJ_I32 = [_as_i32(ox + oy * _P1 + oz * _P2) for (ox, oy, oz) in _OFFS]

NW = 32
PB = 128
NB = 8 * PB
KCH = 123
NPAD = NW * KCH * PB


def _sc_body(pts_hbm, h2v_hbm, feats_hbm, out_hbm,
             ptsv, hv, vid, sv, wb, featbuf, outbuf, sem):
    cid = lax.axis_index("c")
    sid = lax.axis_index("s")
    wid = sid.astype(jnp.int32) * jnp.int32(2) + cid.astype(jnp.int32)

    def chunk_body(k, carry):
        base = (wid * jnp.int32(KCH) + k) * jnp.int32(PB)
        pltpu.sync_copy(pts_hbm.at[:, pl.ds(base, PB)], ptsv)

        def grp(g, c):
            s = g * jnp.int32(L)
            x = ptsv[0, pl.ds(s, L)] * jnp.float32(G)
            y = ptsv[1, pl.ds(s, L)] * jnp.float32(G)
            z = ptsv[2, pl.ds(s, L)] * jnp.float32(G)
            bx = x.astype(jnp.int32)
            by = y.astype(jnp.int32)
            bz = z.astype(jnp.int32)
            fx = x - bx.astype(jnp.float32)
            fy = y - by.astype(jnp.float32)
            fz = z - bz.astype(jnp.float32)
            one = jnp.float32(1.0)
            hb = bx + by * jnp.int32(_P1_I32) + bz * jnp.int32(_P2_I32)
            for j, (ox, oy, oz) in enumerate(_OFFS):
                hv[pl.ds(jnp.int32(j * PB) + s, L)] = (
                    (hb + jnp.int32(_CJ_I32[j])) & jnp.int32(MASK))
                w = ((fx if ox else one - fx)
                     * (fy if oy else one - fy)
                     * (fz if oz else one - fz))
                wb[pl.ds(jnp.int32(j * PB) + s, L)] = w
            return c

        lax.fori_loop(jnp.int32(0), jnp.int32(PB // L), grp, jnp.int32(0))

        pltpu.async_copy(h2v_hbm.at[hv], vid, sem).wait()

        def vgrp(g, c):
            s = g * jnp.int32(L)
            v = vid[pl.ds(s, L)]
            val = v >= 0
            sv[pl.ds(s, L)] = jnp.where(val, v, 0)
            wb[pl.ds(s, L)] = jnp.where(val, wb[pl.ds(s, L)], jnp.float32(0.0))
            return c

        lax.fori_loop(jnp.int32(0), jnp.int32(NB // L), vgrp, jnp.int32(0))

        pltpu.async_copy(feats_hbm.at[sv], featbuf, sem).wait()

        def gblend(g, c):
            s = g * jnp.int32(L)
            wvs = [wb[pl.ds(jnp.int32(j * PB) + s, L)] for j in range(8)]
            for i in range(L):
                p = s + jnp.int32(i)
                acc0 = jnp.zeros((L,), jnp.float32)
                acc1 = jnp.zeros((L,), jnp.float32)
                for j in range(8):
                    w = wvs[j][i]
                    row = featbuf[jnp.int32(j * PB) + p, pl.ds(0, 2 * L)]
                    fe, fo = plsc.unpack(row, format=plsc.PackFormat.INTERLEAVED)
                    acc0 = acc0 + w * fe
                    acc1 = acc1 + w * fo
                outbuf[p, pl.ds(0, L)] = acc0
                outbuf[p, pl.ds(L, L)] = acc1
            return c

        lax.fori_loop(jnp.int32(0), jnp.int32(PB // L), gblend, jnp.int32(0))

        pltpu.sync_copy(outbuf, out_hbm.at[pl.ds(base, PB)])
        return carry

    lax.fori_loop(jnp.int32(0), jnp.int32(KCH), chunk_body, jnp.int32(0))


@jax.jit
def _run(pts_t, h2v, feats):
    mesh = plsc.VectorSubcoreMesh(core_axis_name="c", subcore_axis_name="s",
                                  num_cores=2, num_subcores=16)
    k = pl.kernel(
        _sc_body,
        out_type=jax.ShapeDtypeStruct((NPAD, D), jnp.float32),
        mesh=mesh,
        compiler_params=pltpu.CompilerParams(use_tc_tiling_on_sc=False,
                                            needs_layout_passes=False),
        scratch_types=[
            pltpu.VMEM((3, PB), jnp.float32),
            pltpu.VMEM((NB,), jnp.int32),
            pltpu.VMEM((NB,), jnp.int32),
            pltpu.VMEM((NB,), jnp.int32),
            pltpu.VMEM((NB,), jnp.float32),
            pltpu.VMEM((NB, D), jnp.bfloat16),
            pltpu.VMEM((PB, D), jnp.float32),
            pltpu.SemaphoreType.DMA,
        ],
    )
    return k(pts_t, h2v, feats)


def kernel(pts, voxel_features, hash2vox):
    n = pts.shape[0]
    pad = NPAD - n
    pts_p = jnp.concatenate([pts, jnp.zeros((pad, 3), pts.dtype)], axis=0)
    pts_t = pts_p.T
    h2v = hash2vox.astype(jnp.int32)
    feats = voxel_features.astype(jnp.bfloat16)
    out = _run(pts_t, h2v, feats)[:n]
    return jnp.stack([out[:, :L], out[:, L:]], axis=-1).reshape(n, D)


# speedup vs baseline: 2.7373x; 2.7373x over previous
import jax
import jax.numpy as jnp
from jax import lax
from jax.experimental import pallas as pl
from jax.experimental.pallas import tpu as pltpu
from jax.experimental.pallas import tpu_sc as plsc

G = 128
D = 32
HD = 16
L = 16
MASK = (1 << 21) - 1
_P1 = 2654435761
_P2 = 805459861
V = 262144
VHI = 248000

_OFFS = [(0, 0, 0), (1, 0, 0), (0, 1, 0), (1, 1, 0),
         (0, 0, 1), (1, 0, 1), (0, 1, 1), (1, 1, 1)]


def _as_i32(v):
    v &= 0xFFFFFFFF
    return v - (1 << 32) if v >= (1 << 31) else v


_P1_I32 = _as_i32(_P1)
_P2_I32 = _as_i32(_P2)
_CJ_I32 = [_as_i32(ox + oy * _P1 + oz * _P2) for (ox, oy, oz) in _OFFS]

NS = 16
PB = 64
NB = 8 * PB
KCH = 492
NPAD = NS * KCH * PB


def _sc_body(pts_hbm, h2v_hbm, feats_hbm, out_hbm,
             ptsv, hv, vid, wb, featbuf, oute, outo, shtab, sem):
    cid = lax.axis_index("c").astype(jnp.int32)
    sid = lax.axis_index("s").astype(jnp.int32)

    @pl.when(sid == jnp.int32(0))
    def _():
        pltpu.sync_copy(feats_hbm.at[cid, pl.ds(jnp.int32(0), VHI)], shtab)

    plsc.subcore_barrier()

    lane = lax.broadcasted_iota(jnp.int32, (L,), 0)
    hi8 = lane >= jnp.int32(8)
    hi8i = hi8.astype(jnp.int32)
    lanec = lane & jnp.int32(7)

    def chunk_body(k, carry):
        base = (sid * jnp.int32(KCH) + k) * jnp.int32(PB)
        pltpu.sync_copy(pts_hbm.at[:, pl.ds(base, PB)], ptsv)

        def grp(g, c):
            s = g * jnp.int32(L)
            x = ptsv[0, pl.ds(s, L)] * jnp.float32(G)
            y = ptsv[1, pl.ds(s, L)] * jnp.float32(G)
            z = ptsv[2, pl.ds(s, L)] * jnp.float32(G)
            bx = x.astype(jnp.int32)
            by = y.astype(jnp.int32)
            bz = z.astype(jnp.int32)
            fx = x - bx.astype(jnp.float32)
            fy = y - by.astype(jnp.float32)
            fz = z - bz.astype(jnp.float32)
            one = jnp.float32(1.0)
            hb = bx + by * jnp.int32(_P1_I32) + bz * jnp.int32(_P2_I32)
            for j, (ox, oy, oz) in enumerate(_OFFS):
                hv[pl.ds(jnp.int32(j * PB) + s, L)] = (
                    (hb + jnp.int32(_CJ_I32[j])) & jnp.int32(MASK))
                w = ((fx if ox else one - fx)
                     * (fy if oy else one - fy)
                     * (fz if oz else one - fz))
                wb[pl.ds(jnp.int32(j * PB) + s, L)] = w
            return c

        lax.fori_loop(jnp.int32(0), jnp.int32(PB // L), grp, jnp.int32(0))

        pltpu.async_copy(h2v_hbm.at[hv], vid, sem).wait()

        def vgrp(g, c):
            s = g * jnp.int32(L)
            v = vid[pl.ds(s, L)]
            val = v >= 0
            sv = jnp.where(val, v, 0)
            hv[pl.ds(s, L)] = jnp.minimum(sv, jnp.int32(VHI - 1))
            wb[pl.ds(s, L)] = jnp.where(val, wb[pl.ds(s, L)], jnp.float32(0.0))
            return c

        lax.fori_loop(jnp.int32(0), jnp.int32(NB // L), vgrp, jnp.int32(0))

        pltpu.async_copy(shtab.at[hv], featbuf, sem).wait()

        def pgrp(g, c):
            s = g * jnp.int32(L)
            v = vid[pl.ds(s, L)]
            ov = v >= jnp.int32(VHI)
            cnt = plsc.all_reduce_population_count(ov)

            @pl.when(cnt[0] > jnp.int32(0))
            def _():
                for i in range(L):
                    vi = v[i]

                    @pl.when(vi >= jnp.int32(VHI))
                    def _():
                        pltpu.sync_copy(feats_hbm.at[cid, vi],
                                        featbuf.at[s + jnp.int32(i)])
            return c

        lax.fori_loop(jnp.int32(0), jnp.int32(NB // L), pgrp, jnp.int32(0))

        def gblend(g, c):
            s = g * jnp.int32(L)
            wvs = [wb[pl.ds(jnp.int32(j * PB) + s, L)] for j in range(8)]
            for i in range(0, L, 2):
                p = s + jnp.int32(i)
                acc0 = jnp.zeros((L,), jnp.float32)
                acc1 = jnp.zeros((L,), jnp.float32)
                for j in range(8):
                    idxr = jnp.int32(j * PB) + p + hi8i
                    pair = plsc.load_gather(featbuf, [idxr, lanec])
                    bits = plsc.bitcast(pair, jnp.bfloat16)
                    fe, fo = plsc.unpack(bits,
                                         format=plsc.PackFormat.INTERLEAVED)
                    w2 = jnp.where(hi8, wvs[j][i + 1], wvs[j][i])
                    acc0 = acc0 + w2 * fe
                    acc1 = acc1 + w2 * fo
                oute[pl.ds(p * jnp.int32(8), L)] = acc0
                outo[pl.ds(p * jnp.int32(8), L)] = acc1
            return c

        lax.fori_loop(jnp.int32(0), jnp.int32(PB // L), gblend, jnp.int32(0))

        pltpu.sync_copy(
            oute, out_hbm.at[cid, jnp.int32(0),
                             pl.ds(base * jnp.int32(8), PB * 8)])
        pltpu.sync_copy(
            outo, out_hbm.at[cid, jnp.int32(1),
                             pl.ds(base * jnp.int32(8), PB * 8)])
        return carry

    lax.fori_loop(jnp.int32(0), jnp.int32(KCH), chunk_body, jnp.int32(0))


@jax.jit
def _run(pts_t, h2v, feats_b):
    mesh = plsc.VectorSubcoreMesh(core_axis_name="c", subcore_axis_name="s",
                                  num_cores=2, num_subcores=16)
    k = pl.kernel(
        _sc_body,
        out_type=jax.ShapeDtypeStruct((2, 2, NPAD * 8), jnp.float32),
        mesh=mesh,
        compiler_params=pltpu.CompilerParams(use_tc_tiling_on_sc=False,
                                            needs_layout_passes=False),
        scratch_types=[
            pltpu.VMEM((3, PB), jnp.float32),
            pltpu.VMEM((NB,), jnp.int32),
            pltpu.VMEM((NB,), jnp.int32),
            pltpu.VMEM((NB,), jnp.float32),
            pltpu.VMEM((NB, 8), jnp.int32),
            pltpu.VMEM((PB * 8,), jnp.float32),
            pltpu.VMEM((PB * 8,), jnp.float32),
            pltpu.VMEM_SHARED((VHI, 8), jnp.int32),
            pltpu.SemaphoreType.DMA,
        ],
    )
    return k(pts_t, h2v, feats_b)


def kernel(pts, voxel_features, hash2vox):
    n = pts.shape[0]
    pad = NPAD - n
    pts_p = jnp.concatenate([pts, jnp.zeros((pad, 3), pts.dtype)], axis=0)
    pts_t = pts_p.T
    h2v = hash2vox.astype(jnp.int32)
    fb = voxel_features.astype(jnp.bfloat16)
    fbs = jnp.stack([fb[:, :HD], fb[:, HD:]], axis=0)
    feats_b = lax.bitcast_convert_type(
        fbs.reshape(2, V, 8, 2), jnp.int32)
    out = _run(pts_t, h2v, feats_b)
    o = out.reshape(2, 2, NPAD, 8)[:, :, :n, :]
    h0 = jnp.stack([o[0, 0], o[0, 1]], axis=-1).reshape(n, HD)
    h1 = jnp.stack([o[1, 0], o[1, 1]], axis=-1).reshape(n, HD)
    return jnp.concatenate([h0, h1], axis=1)


# speedup vs baseline: 3.2412x; 1.1841x over previous
import jax
import jax.numpy as jnp
from jax import lax
from jax.experimental import pallas as pl
from jax.experimental.pallas import tpu as pltpu
from jax.experimental.pallas import tpu_sc as plsc

G = 128
D = 32
HD = 16
L = 16
MASK = (1 << 21) - 1
_P1 = 2654435761
_P2 = 805459861
V = 262144
VHI = 248000

_OFFS = [(0, 0, 0), (1, 0, 0), (0, 1, 0), (1, 1, 0),
         (0, 0, 1), (1, 0, 1), (0, 1, 1), (1, 1, 1)]


def _as_i32(v):
    v &= 0xFFFFFFFF
    return v - (1 << 32) if v >= (1 << 31) else v


_P1_I32 = _as_i32(_P1)
_P2_I32 = _as_i32(_P2)
_CJ_I32 = [_as_i32(ox + oy * _P1 + oz * _P2) for (ox, oy, oz) in _OFFS]

NS = 16
PB = 64
NB = 8 * PB
KCH = 492
NPAD = NS * KCH * PB


def _sc_body(pts_hbm, h2v_hbm, feats_hbm, out_hbm,
             ptsv, hv, vid, wb, featbuf, oute, outo, shtab, sem, sem2):
    cid = lax.axis_index("c").astype(jnp.int32)
    sid = lax.axis_index("s").astype(jnp.int32)

    @pl.when(sid == jnp.int32(0))
    def _():
        pltpu.sync_copy(feats_hbm.at[cid, pl.ds(jnp.int32(0), VHI)], shtab)

    plsc.subcore_barrier()

    lane = lax.broadcasted_iota(jnp.int32, (L,), 0)
    hi8 = lane >= jnp.int32(8)
    hi8i = hi8.astype(jnp.int32)
    lanec = lane & jnp.int32(7)

    def chunk_body(k, carry):
        base = (sid * jnp.int32(KCH) + k) * jnp.int32(PB)
        pltpu.sync_copy(pts_hbm.at[:, pl.ds(base, PB)], ptsv)

        def grp(g, c):
            s = g * jnp.int32(L)
            x = ptsv[0, pl.ds(s, L)] * jnp.float32(G)
            y = ptsv[1, pl.ds(s, L)] * jnp.float32(G)
            z = ptsv[2, pl.ds(s, L)] * jnp.float32(G)
            bx = x.astype(jnp.int32)
            by = y.astype(jnp.int32)
            bz = z.astype(jnp.int32)
            fx = x - bx.astype(jnp.float32)
            fy = y - by.astype(jnp.float32)
            fz = z - bz.astype(jnp.float32)
            one = jnp.float32(1.0)
            hb = bx + by * jnp.int32(_P1_I32) + bz * jnp.int32(_P2_I32)
            for j, (ox, oy, oz) in enumerate(_OFFS):
                hv[pl.ds(jnp.int32(j * PB) + s, L)] = (
                    (hb + jnp.int32(_CJ_I32[j])) & jnp.int32(MASK))
                w = ((fx if ox else one - fx)
                     * (fy if oy else one - fy)
                     * (fz if oz else one - fz))
                wb[pl.ds(jnp.int32(j * PB) + s, L)] = w
            return c

        lax.fori_loop(jnp.int32(0), jnp.int32(PB // L), grp, jnp.int32(0))

        pltpu.async_copy(h2v_hbm.at[hv], vid, sem).wait()

        def vgrp(g, c):
            s = g * jnp.int32(L)
            v = vid[pl.ds(s, L)]
            val = v >= 0
            sv = jnp.where(val, v, 0)
            hv[pl.ds(s, L)] = jnp.minimum(sv, jnp.int32(VHI - 1))
            wb[pl.ds(s, L)] = jnp.where(val, wb[pl.ds(s, L)], jnp.float32(0.0))
            return c

        lax.fori_loop(jnp.int32(0), jnp.int32(NB // L), vgrp, jnp.int32(0))

        pltpu.async_copy(shtab.at[hv], featbuf, sem).wait()

        def pgrp(g, c):
            s = g * jnp.int32(L)
            v = vid[pl.ds(s, L)]
            ov = v >= jnp.int32(VHI)
            cnt = plsc.all_reduce_population_count(ov)

            @pl.when(cnt[0] > jnp.int32(0))
            def _():
                for i in range(L):
                    vi = v[i]

                    @pl.when(vi >= jnp.int32(VHI))
                    def _():
                        pltpu.async_copy(feats_hbm.at[cid, vi],
                                         featbuf.at[s + jnp.int32(i)], sem2)
            return c + cnt[0]

        npatch = lax.fori_loop(jnp.int32(0), jnp.int32(NB // L), pgrp,
                               jnp.int32(0))

        def pdrain(i, c):
            pltpu.make_async_copy(feats_hbm.at[cid, jnp.int32(0)],
                                  featbuf.at[jnp.int32(0)], sem2).wait()
            return c

        lax.fori_loop(jnp.int32(0), npatch, pdrain, jnp.int32(0))

        def gblend(g, c):
            s = g * jnp.int32(L)
            wvs = [wb[pl.ds(jnp.int32(j * PB) + s, L)] for j in range(8)]
            for i in range(0, L, 2):
                p = s + jnp.int32(i)
                acc0 = jnp.zeros((L,), jnp.float32)
                acc1 = jnp.zeros((L,), jnp.float32)
                for j in range(8):
                    idxr = jnp.int32(j * PB) + p + hi8i
                    pair = plsc.load_gather(featbuf, [idxr, lanec])
                    bits = plsc.bitcast(pair, jnp.bfloat16)
                    fe, fo = plsc.unpack(bits,
                                         format=plsc.PackFormat.INTERLEAVED)
                    w2 = jnp.where(hi8, wvs[j][i + 1], wvs[j][i])
                    acc0 = acc0 + w2 * fe
                    acc1 = acc1 + w2 * fo
                oute[pl.ds(p * jnp.int32(8), L)] = acc0
                outo[pl.ds(p * jnp.int32(8), L)] = acc1
            return c

        lax.fori_loop(jnp.int32(0), jnp.int32(PB // L), gblend, jnp.int32(0))

        pltpu.sync_copy(
            oute, out_hbm.at[cid, jnp.int32(0),
                             pl.ds(base * jnp.int32(8), PB * 8)])
        pltpu.sync_copy(
            outo, out_hbm.at[cid, jnp.int32(1),
                             pl.ds(base * jnp.int32(8), PB * 8)])
        return carry

    lax.fori_loop(jnp.int32(0), jnp.int32(KCH), chunk_body, jnp.int32(0))


@jax.jit
def _run(pts_t, h2v, feats_b):
    mesh = plsc.VectorSubcoreMesh(core_axis_name="c", subcore_axis_name="s",
                                  num_cores=2, num_subcores=16)
    k = pl.kernel(
        _sc_body,
        out_type=jax.ShapeDtypeStruct((2, 2, NPAD * 8), jnp.float32),
        mesh=mesh,
        compiler_params=pltpu.CompilerParams(use_tc_tiling_on_sc=False,
                                            needs_layout_passes=False),
        scratch_types=[
            pltpu.VMEM((3, PB), jnp.float32),
            pltpu.VMEM((NB,), jnp.int32),
            pltpu.VMEM((NB,), jnp.int32),
            pltpu.VMEM((NB,), jnp.float32),
            pltpu.VMEM((NB, 8), jnp.int32),
            pltpu.VMEM((PB * 8,), jnp.float32),
            pltpu.VMEM((PB * 8,), jnp.float32),
            pltpu.VMEM_SHARED((VHI, 8), jnp.int32),
            pltpu.SemaphoreType.DMA,
            pltpu.SemaphoreType.DMA,
        ],
    )
    return k(pts_t, h2v, feats_b)


def kernel(pts, voxel_features, hash2vox):
    n = pts.shape[0]
    pad = NPAD - n
    pts_p = jnp.concatenate([pts, jnp.zeros((pad, 3), pts.dtype)], axis=0)
    pts_t = pts_p.T
    h2v = hash2vox.astype(jnp.int32)
    fb = voxel_features.astype(jnp.bfloat16)
    fbs = jnp.stack([fb[:, :HD], fb[:, HD:]], axis=0)
    feats_b = lax.bitcast_convert_type(
        fbs.reshape(2, V, 8, 2), jnp.int32)
    out = _run(pts_t, h2v, feats_b)
    o = out.reshape(2, 2, NPAD, 8)[:, :, :n, :]
    h0 = jnp.stack([o[0, 0], o[0, 1]], axis=-1).reshape(n, HD)
    h1 = jnp.stack([o[1, 0], o[1, 1]], axis=-1).reshape(n, HD)
    return jnp.concatenate([h0, h1], axis=1)


# speedup vs baseline: 3.4354x; 1.0599x over previous
import jax
import jax.numpy as jnp
from jax import lax
from jax.experimental import pallas as pl
from jax.experimental.pallas import tpu as pltpu
from jax.experimental.pallas import tpu_sc as plsc

G = 128
D = 32
HD = 16
L = 16
MASK = (1 << 21) - 1
_P1 = 2654435761
_P2 = 805459861
V = 262144
VHI = 244800

_OFFS = [(0, 0, 0), (1, 0, 0), (0, 1, 0), (1, 1, 0),
         (0, 0, 1), (1, 0, 1), (0, 1, 1), (1, 1, 1)]


def _as_i32(v):
    v &= 0xFFFFFFFF
    return v - (1 << 32) if v >= (1 << 31) else v


_P1_I32 = _as_i32(_P1)
_P2_I32 = _as_i32(_P2)
_CJ_I32 = [_as_i32(ox + oy * _P1 + oz * _P2) for (ox, oy, oz) in _OFFS]

NS = 16
PB = 64
NB = 8 * PB
KCH = 492
NPAD = NS * KCH * PB


def _sc_body(pts_hbm, h2v_hbm, feats_hbm, out_hbm,
             ptsv, hv, vid, wb, ptsv2, hv2, vid2, wb2,
             featbuf, oute, outo, shtab, sem, sem2, semA, semB):
    cid = lax.axis_index("c").astype(jnp.int32)
    sid = lax.axis_index("s").astype(jnp.int32)

    @pl.when(sid == jnp.int32(0))
    def _():
        pltpu.sync_copy(feats_hbm.at[cid, pl.ds(jnp.int32(0), VHI)], shtab)

    plsc.subcore_barrier()

    lane = lax.broadcasted_iota(jnp.int32, (L,), 0)
    hi8 = lane >= jnp.int32(8)
    hi8i = hi8.astype(jnp.int32)
    lanec = lane & jnp.int32(7)

    def load_and_hash(k, ptsv_r, hv_r, vid_r, wb_r, hsem):
        base = (sid * jnp.int32(KCH) + k) * jnp.int32(PB)
        pltpu.sync_copy(pts_hbm.at[:, pl.ds(base, PB)], ptsv_r)

        def grp(g, c):
            s = g * jnp.int32(L)
            x = ptsv_r[0, pl.ds(s, L)] * jnp.float32(G)
            y = ptsv_r[1, pl.ds(s, L)] * jnp.float32(G)
            z = ptsv_r[2, pl.ds(s, L)] * jnp.float32(G)
            bx = x.astype(jnp.int32)
            by = y.astype(jnp.int32)
            bz = z.astype(jnp.int32)
            fx = x - bx.astype(jnp.float32)
            fy = y - by.astype(jnp.float32)
            fz = z - bz.astype(jnp.float32)
            one = jnp.float32(1.0)
            hb = bx + by * jnp.int32(_P1_I32) + bz * jnp.int32(_P2_I32)
            for j, (ox, oy, oz) in enumerate(_OFFS):
                hv_r[pl.ds(jnp.int32(j * PB) + s, L)] = (
                    (hb + jnp.int32(_CJ_I32[j])) & jnp.int32(MASK))
                w = ((fx if ox else one - fx)
                     * (fy if oy else one - fy)
                     * (fz if oz else one - fz))
                wb_r[pl.ds(jnp.int32(j * PB) + s, L)] = w
            return c

        lax.fori_loop(jnp.int32(0), jnp.int32(PB // L), grp, jnp.int32(0))
        pltpu.async_copy(h2v_hbm.at[hv_r], vid_r, hsem)

    def process(k, hv_r, vid_r, wb_r, hsem):
        base = (sid * jnp.int32(KCH) + k) * jnp.int32(PB)
        pltpu.make_async_copy(h2v_hbm.at[hv_r], vid_r, hsem).wait()

        def vgrp(g, c):
            s = g * jnp.int32(L)
            v = vid_r[pl.ds(s, L)]
            val = v >= 0
            sv = jnp.where(val, v, 0)
            hv_r[pl.ds(s, L)] = jnp.minimum(sv, jnp.int32(VHI - 1))
            wb_r[pl.ds(s, L)] = jnp.where(val, wb_r[pl.ds(s, L)],
                                          jnp.float32(0.0))
            return c

        lax.fori_loop(jnp.int32(0), jnp.int32(NB // L), vgrp, jnp.int32(0))

        pltpu.async_copy(shtab.at[hv_r], featbuf, sem).wait()

        def pgrp(g, c):
            s = g * jnp.int32(L)
            v = vid_r[pl.ds(s, L)]
            ov = v >= jnp.int32(VHI)
            cnt = plsc.all_reduce_population_count(ov)

            @pl.when(cnt[0] > jnp.int32(0))
            def _():
                for i in range(L):
                    vi = v[i]

                    @pl.when(vi >= jnp.int32(VHI))
                    def _():
                        pltpu.async_copy(feats_hbm.at[cid, vi],
                                         featbuf.at[s + jnp.int32(i)], sem2)
            return c + cnt[0]

        npatch = lax.fori_loop(jnp.int32(0), jnp.int32(NB // L), pgrp,
                               jnp.int32(0))

        def pdrain(i, c):
            pltpu.make_async_copy(feats_hbm.at[cid, jnp.int32(0)],
                                  featbuf.at[jnp.int32(0)], sem2).wait()
            return c

        lax.fori_loop(jnp.int32(0), npatch, pdrain, jnp.int32(0))

        def gblend(g, c):
            s = g * jnp.int32(L)
            wvs = [wb_r[pl.ds(jnp.int32(j * PB) + s, L)] for j in range(8)]
            for i in range(0, L, 2):
                p = s + jnp.int32(i)
                acc0 = jnp.zeros((L,), jnp.float32)
                acc1 = jnp.zeros((L,), jnp.float32)
                for j in range(8):
                    idxr = jnp.int32(j * PB) + p + hi8i
                    pair = plsc.load_gather(featbuf, [idxr, lanec])
                    bits = plsc.bitcast(pair, jnp.bfloat16)
                    fe, fo = plsc.unpack(bits,
                                         format=plsc.PackFormat.INTERLEAVED)
                    w2 = jnp.where(hi8, wvs[j][i + 1], wvs[j][i])
                    acc0 = acc0 + w2 * fe
                    acc1 = acc1 + w2 * fo
                oute[pl.ds(p * jnp.int32(8), L)] = acc0
                outo[pl.ds(p * jnp.int32(8), L)] = acc1
            return c

        lax.fori_loop(jnp.int32(0), jnp.int32(PB // L), gblend, jnp.int32(0))

        pltpu.sync_copy(
            oute, out_hbm.at[cid, jnp.int32(0),
                             pl.ds(base * jnp.int32(8), PB * 8)])
        pltpu.sync_copy(
            outo, out_hbm.at[cid, jnp.int32(1),
                             pl.ds(base * jnp.int32(8), PB * 8)])

    load_and_hash(jnp.int32(0), ptsv, hv, vid, wb, semA)

    def pipe(k2, carry):
        c0 = k2 * jnp.int32(2)
        load_and_hash(c0 + jnp.int32(1), ptsv2, hv2, vid2, wb2, semB)
        process(c0, hv, vid, wb, semA)

        @pl.when(c0 + jnp.int32(2) < jnp.int32(KCH))
        def _():
            load_and_hash(c0 + jnp.int32(2), ptsv, hv, vid, wb, semA)

        process(c0 + jnp.int32(1), hv2, vid2, wb2, semB)
        return carry

    lax.fori_loop(jnp.int32(0), jnp.int32(KCH // 2), pipe, jnp.int32(0))


@jax.jit
def _run(pts_t, h2v, feats_b):
    mesh = plsc.VectorSubcoreMesh(core_axis_name="c", subcore_axis_name="s",
                                  num_cores=2, num_subcores=16)
    k = pl.kernel(
        _sc_body,
        out_type=jax.ShapeDtypeStruct((2, 2, NPAD * 8), jnp.float32),
        mesh=mesh,
        compiler_params=pltpu.CompilerParams(use_tc_tiling_on_sc=False,
                                            needs_layout_passes=False),
        scratch_types=[
            pltpu.VMEM((3, PB), jnp.float32),
            pltpu.VMEM((NB,), jnp.int32),
            pltpu.VMEM((NB,), jnp.int32),
            pltpu.VMEM((NB,), jnp.float32),
            pltpu.VMEM((3, PB), jnp.float32),
            pltpu.VMEM((NB,), jnp.int32),
            pltpu.VMEM((NB,), jnp.int32),
            pltpu.VMEM((NB,), jnp.float32),
            pltpu.VMEM((NB, 8), jnp.int32),
            pltpu.VMEM((PB * 8,), jnp.float32),
            pltpu.VMEM((PB * 8,), jnp.float32),
            pltpu.VMEM_SHARED((VHI, 8), jnp.int32),
            pltpu.SemaphoreType.DMA,
            pltpu.SemaphoreType.DMA,
            pltpu.SemaphoreType.DMA,
            pltpu.SemaphoreType.DMA,
        ],
    )
    return k(pts_t, h2v, feats_b)


def kernel(pts, voxel_features, hash2vox):
    n = pts.shape[0]
    pad = NPAD - n
    pts_p = jnp.concatenate([pts, jnp.zeros((pad, 3), pts.dtype)], axis=0)
    pts_t = pts_p.T
    h2v = hash2vox.astype(jnp.int32)
    fb = voxel_features.astype(jnp.bfloat16)
    fbs = jnp.stack([fb[:, :HD], fb[:, HD:]], axis=0)
    feats_b = lax.bitcast_convert_type(
        fbs.reshape(2, V, 8, 2), jnp.int32)
    out = _run(pts_t, h2v, feats_b)
    o = out.reshape(2, 2, NPAD, 8)[:, :, :n, :]
    h0 = jnp.stack([o[0, 0], o[0, 1]], axis=-1).reshape(n, HD)
    h1 = jnp.stack([o[1, 0], o[1, 1]], axis=-1).reshape(n, HD)
    return jnp.concatenate([h0, h1], axis=1)


# speedup vs baseline: 3.4537x; 1.0053x over previous
import jax
import jax.numpy as jnp
from jax import lax
from jax.experimental import pallas as pl
from jax.experimental.pallas import tpu as pltpu
from jax.experimental.pallas import tpu_sc as plsc

G = 128
D = 32
HD = 16
L = 16
MASK = (1 << 21) - 1
_P1 = 2654435761
_P2 = 805459861
V = 262144
VHI = 236608

_OFFS = [(0, 0, 0), (1, 0, 0), (0, 1, 0), (1, 1, 0),
         (0, 0, 1), (1, 0, 1), (0, 1, 1), (1, 1, 1)]


def _as_i32(v):
    v &= 0xFFFFFFFF
    return v - (1 << 32) if v >= (1 << 31) else v


_P1_I32 = _as_i32(_P1)
_P2_I32 = _as_i32(_P2)
_CJ_I32 = [_as_i32(ox + oy * _P1 + oz * _P2) for (ox, oy, oz) in _OFFS]

NS = 16
PB = 64
NB = 8 * PB
KCH = 492
NPAD = NS * KCH * PB


def _sc_body(pts_hbm, h2v_hbm, feats_hbm, out_hbm,
             ptsv, hv, vid, wb, ptsv2, hv2, vid2, wb2,
             featbuf, featbuf2, oute, outo, shtab, sem, sem2, semA, semB,
             sem3):
    cid = lax.axis_index("c").astype(jnp.int32)
    sid = lax.axis_index("s").astype(jnp.int32)

    @pl.when(sid == jnp.int32(0))
    def _():
        pltpu.sync_copy(feats_hbm.at[cid, pl.ds(jnp.int32(0), VHI)], shtab)

    plsc.subcore_barrier()

    lane = lax.broadcasted_iota(jnp.int32, (L,), 0)
    hi8 = lane >= jnp.int32(8)
    hi8i = hi8.astype(jnp.int32)
    lanec = lane & jnp.int32(7)

    def load_and_hash(k, ptsv_r, hv_r, vid_r, wb_r, hsem):
        base = (sid * jnp.int32(KCH) + k) * jnp.int32(PB)
        pltpu.sync_copy(pts_hbm.at[:, pl.ds(base, PB)], ptsv_r)

        def grp(g, c):
            s = g * jnp.int32(L)
            x = ptsv_r[0, pl.ds(s, L)] * jnp.float32(G)
            y = ptsv_r[1, pl.ds(s, L)] * jnp.float32(G)
            z = ptsv_r[2, pl.ds(s, L)] * jnp.float32(G)
            bx = x.astype(jnp.int32)
            by = y.astype(jnp.int32)
            bz = z.astype(jnp.int32)
            fx = x - bx.astype(jnp.float32)
            fy = y - by.astype(jnp.float32)
            fz = z - bz.astype(jnp.float32)
            one = jnp.float32(1.0)
            hb = bx + by * jnp.int32(_P1_I32) + bz * jnp.int32(_P2_I32)
            for j, (ox, oy, oz) in enumerate(_OFFS):
                hv_r[pl.ds(jnp.int32(j * PB) + s, L)] = (
                    (hb + jnp.int32(_CJ_I32[j])) & jnp.int32(MASK))
                w = ((fx if ox else one - fx)
                     * (fy if oy else one - fy)
                     * (fz if oz else one - fz))
                wb_r[pl.ds(jnp.int32(j * PB) + s, L)] = w
            return c

        lax.fori_loop(jnp.int32(0), jnp.int32(PB // L), grp, jnp.int32(0))
        pltpu.async_copy(h2v_hbm.at[hv_r], vid_r, hsem)

    def pre(k, hv_r, vid_r, wb_r, fb_r, hsem, ssem):
        pltpu.make_async_copy(h2v_hbm.at[hv_r], vid_r, hsem).wait()

        def vgrp(g, c):
            s = g * jnp.int32(L)
            v = vid_r[pl.ds(s, L)]
            val = v >= 0
            sv = jnp.where(val, v, 0)
            hv_r[pl.ds(s, L)] = jnp.minimum(sv, jnp.int32(VHI - 1))
            wb_r[pl.ds(s, L)] = jnp.where(val, wb_r[pl.ds(s, L)],
                                          jnp.float32(0.0))
            return c

        lax.fori_loop(jnp.int32(0), jnp.int32(NB // L), vgrp, jnp.int32(0))

        pltpu.async_copy(shtab.at[hv_r], fb_r, ssem)

    def post(k, hv_r, vid_r, wb_r, fb_r, ssem):
        base = (sid * jnp.int32(KCH) + k) * jnp.int32(PB)
        pltpu.make_async_copy(shtab.at[hv_r], fb_r, ssem).wait()

        def pgrp(g, c):
            s = g * jnp.int32(L)
            v = vid_r[pl.ds(s, L)]
            ov = v >= jnp.int32(VHI)
            cnt = plsc.all_reduce_population_count(ov)

            @pl.when(cnt[0] > jnp.int32(0))
            def _():
                for i in range(L):
                    vi = v[i]

                    @pl.when(vi >= jnp.int32(VHI))
                    def _():
                        pltpu.async_copy(feats_hbm.at[cid, vi],
                                         fb_r.at[s + jnp.int32(i)], sem2)
            return c + cnt[0]

        npatch = lax.fori_loop(jnp.int32(0), jnp.int32(NB // L), pgrp,
                               jnp.int32(0))

        def pdrain(i, c):
            pltpu.make_async_copy(feats_hbm.at[cid, jnp.int32(0)],
                                  fb_r.at[jnp.int32(0)], sem2).wait()
            return c

        lax.fori_loop(jnp.int32(0), npatch, pdrain, jnp.int32(0))

        def gblend(g, c):
            s = g * jnp.int32(L)
            wvs = [wb_r[pl.ds(jnp.int32(j * PB) + s, L)] for j in range(8)]
            for i in range(0, L, 2):
                p = s + jnp.int32(i)
                acc0 = jnp.zeros((L,), jnp.float32)
                acc1 = jnp.zeros((L,), jnp.float32)
                for j in range(8):
                    idxr = jnp.int32(j * PB) + p + hi8i
                    pair = plsc.load_gather(fb_r, [idxr, lanec])
                    bits = plsc.bitcast(pair, jnp.bfloat16)
                    fe, fo = plsc.unpack(bits,
                                         format=plsc.PackFormat.INTERLEAVED)
                    w2 = jnp.where(hi8, wvs[j][i + 1], wvs[j][i])
                    acc0 = acc0 + w2 * fe
                    acc1 = acc1 + w2 * fo
                oute[pl.ds(p * jnp.int32(8), L)] = acc0
                outo[pl.ds(p * jnp.int32(8), L)] = acc1
            return c

        lax.fori_loop(jnp.int32(0), jnp.int32(PB // L), gblend, jnp.int32(0))

        pltpu.sync_copy(
            oute, out_hbm.at[cid, jnp.int32(0),
                             pl.ds(base * jnp.int32(8), PB * 8)])
        pltpu.sync_copy(
            outo, out_hbm.at[cid, jnp.int32(1),
                             pl.ds(base * jnp.int32(8), PB * 8)])

    load_and_hash(jnp.int32(0), ptsv, hv, vid, wb, semA)

    def pipe(k2, carry):
        c0 = k2 * jnp.int32(2)
        c1 = c0 + jnp.int32(1)
        pre(c0, hv, vid, wb, featbuf, semA, sem)
        load_and_hash(c1, ptsv2, hv2, vid2, wb2, semB)
        post(c0, hv, vid, wb, featbuf, sem)
        pre(c1, hv2, vid2, wb2, featbuf2, semB, sem3)

        @pl.when(c0 + jnp.int32(2) < jnp.int32(KCH))
        def _():
            load_and_hash(c0 + jnp.int32(2), ptsv, hv, vid, wb, semA)

        post(c1, hv2, vid2, wb2, featbuf2, sem3)
        return carry

    lax.fori_loop(jnp.int32(0), jnp.int32(KCH // 2), pipe, jnp.int32(0))


@jax.jit
def _run(pts_t, h2v, feats_b):
    mesh = plsc.VectorSubcoreMesh(core_axis_name="c", subcore_axis_name="s",
                                  num_cores=2, num_subcores=16)
    k = pl.kernel(
        _sc_body,
        out_type=jax.ShapeDtypeStruct((2, 2, NPAD * 8), jnp.float32),
        mesh=mesh,
        compiler_params=pltpu.CompilerParams(use_tc_tiling_on_sc=False,
                                            needs_layout_passes=False),
        scratch_types=[
            pltpu.VMEM((3, PB), jnp.float32),
            pltpu.VMEM((NB,), jnp.int32),
            pltpu.VMEM((NB,), jnp.int32),
            pltpu.VMEM((NB,), jnp.float32),
            pltpu.VMEM((3, PB), jnp.float32),
            pltpu.VMEM((NB,), jnp.int32),
            pltpu.VMEM((NB,), jnp.int32),
            pltpu.VMEM((NB,), jnp.float32),
            pltpu.VMEM((NB, 8), jnp.int32),
            pltpu.VMEM((NB, 8), jnp.int32),
            pltpu.VMEM((PB * 8,), jnp.float32),
            pltpu.VMEM((PB * 8,), jnp.float32),
            pltpu.VMEM_SHARED((VHI, 8), jnp.int32),
            pltpu.SemaphoreType.DMA,
            pltpu.SemaphoreType.DMA,
            pltpu.SemaphoreType.DMA,
            pltpu.SemaphoreType.DMA,
            pltpu.SemaphoreType.DMA,
        ],
    )
    return k(pts_t, h2v, feats_b)


def kernel(pts, voxel_features, hash2vox):
    n = pts.shape[0]
    pad = NPAD - n
    pts_p = jnp.concatenate([pts, jnp.zeros((pad, 3), pts.dtype)], axis=0)
    pts_t = pts_p.T
    h2v = hash2vox.astype(jnp.int32)
    fb = voxel_features.astype(jnp.bfloat16)
    fbs = jnp.stack([fb[:, :HD], fb[:, HD:]], axis=0)
    feats_b = lax.bitcast_convert_type(
        fbs.reshape(2, V, 8, 2), jnp.int32)
    out = _run(pts_t, h2v, feats_b)
    o = out.reshape(2, 2, NPAD, 8)[:, :, :n, :]
    h0 = jnp.stack([o[0, 0], o[0, 1]], axis=-1).reshape(n, HD)
    h1 = jnp.stack([o[1, 0], o[1, 1]], axis=-1).reshape(n, HD)
    return jnp.concatenate([h0, h1], axis=1)
